# Initial kernel scaffold; baseline (speedup 1.0000x reference)
import jax
import jax.numpy as jnp
from jax.experimental import pallas as pl


def kernel(pred, target, source):
    raise NotImplementedError("write your pallas kernel here")


# speedup vs baseline: 14.7435x; 14.7435x over previous
import jax
import jax.numpy as jnp
from jax.experimental import pallas as pl
from jax.experimental.pallas import tpu as pltpu

_B, _H, _W = 16, 512, 512
_N = _B * _H * _W
_TOPK = int(_N * 0.1)
_NTHR = 16
_NPASS = 7
_ROWS = _N // _W
_BLK = 512
_NBLK = _ROWS // _BLK


def _edge_stats_kernel(p_ref, t_ref, ep_ref, et_ref, acc_ref):
    i = pl.program_id(0)

    @pl.when(i == 0)
    def _():
        acc_ref[...] = jnp.zeros_like(acc_ref)

    def edges(a):
        z_row = jnp.zeros((1, _W), jnp.float32)
        up = jnp.concatenate([z_row, a[:-1, :]], axis=0)
        dn = jnp.concatenate([a[1:, :], z_row], axis=0)
        s = up + 2.0 * a + dn
        d = dn - up
        z_col = jnp.zeros((_H, 1), jnp.float32)
        ex = jnp.concatenate([s[:, 1:], z_col], axis=1) - \
            jnp.concatenate([z_col, s[:, :-1]], axis=1)
        ey = jnp.concatenate([z_col, d[:, :-1]], axis=1) + 2.0 * d + \
            jnp.concatenate([d[:, 1:], z_col], axis=1)
        return jnp.sqrt(ex * ex + ey * ey + 1e-6)

    ep = edges(p_ref[0])
    et = edges(t_ref[0])
    ep_ref[0] = ep
    et_ref[0] = et

    def lanesum(x):
        return jnp.sum(x.reshape(_H // 8, 8, _W), axis=0)

    acc_ref[0] += lanesum(ep)
    acc_ref[1] += lanesum(ep * ep)
    acc_ref[2] += lanesum(et)
    acc_ref[3] += lanesum(et * et)


def _count_kernel(thr_ref, ep_ref, et_ref, cnt_ref, sm_ref):
    b = pl.program_id(0)

    @pl.when(b == 0)
    def _():
        cnt_ref[...] = jnp.zeros_like(cnt_ref)
        sm_ref[...] = jnp.zeros_like(sm_ref)

    def lanesum(x):
        return jnp.sum(x.reshape(_BLK // 8, 8, _W), axis=0)

    for i, ref in enumerate((ep_ref, et_ref)):
        x = ref[...]
        for j in range(_NTHR):
            t = thr_ref[i, j]
            mask = x > t
            cnt_ref[i, j] += lanesum(mask.astype(jnp.float32))
            sm_ref[i, j] += lanesum(jnp.where(mask, x, 0.0))


def _run_edges(p, t):
    return pl.pallas_call(
        _edge_stats_kernel,
        grid=(_B,),
        in_specs=[
            pl.BlockSpec((1, _H, _W), lambda i: (i, 0, 0)),
            pl.BlockSpec((1, _H, _W), lambda i: (i, 0, 0)),
        ],
        out_specs=[
            pl.BlockSpec((1, _H, _W), lambda i: (i, 0, 0)),
            pl.BlockSpec((1, _H, _W), lambda i: (i, 0, 0)),
            pl.BlockSpec((4, 8, _W), lambda i: (0, 0, 0)),
        ],
        out_shape=[
            jax.ShapeDtypeStruct((_B, _H, _W), jnp.float32),
            jax.ShapeDtypeStruct((_B, _H, _W), jnp.float32),
            jax.ShapeDtypeStruct((4, 8, _W), jnp.float32),
        ],
    )(p, t)


def _run_count(thr, e2p, e2t):
    return pl.pallas_call(
        _count_kernel,
        grid=(_NBLK,),
        in_specs=[
            pl.BlockSpec(memory_space=pltpu.SMEM),
            pl.BlockSpec((_BLK, _W), lambda b: (b, 0)),
            pl.BlockSpec((_BLK, _W), lambda b: (b, 0)),
        ],
        out_specs=[
            pl.BlockSpec((2, _NTHR, 8, _W), lambda b: (0, 0, 0, 0)),
            pl.BlockSpec((2, _NTHR, 8, _W), lambda b: (0, 0, 0, 0)),
        ],
        out_shape=[
            jax.ShapeDtypeStruct((2, _NTHR, 8, _W), jnp.float32),
            jax.ShapeDtypeStruct((2, _NTHR, 8, _W), jnp.float32),
        ],
    )(thr, e2p, e2t)


def kernel(pred, target, source):
    p = pred.reshape(_B, _H, _W)
    t = target.reshape(_B, _H, _W)
    ep, et, acc = _run_edges(p, t)

    sums = jnp.sum(acc, axis=(1, 2))
    n_f = jnp.float32(_N)
    mean_p, mean_t = sums[0] / n_f, sums[2] / n_f
    var_p = (sums[1] - sums[0] * mean_p) / (n_f - 1.0)
    var_t = (sums[3] - sums[2] * mean_t) / (n_f - 1.0)
    stats_loss = jnp.abs(mean_p - mean_t) + jnp.abs(
        jnp.sqrt(var_p) - jnp.sqrt(var_t))

    e2p = ep.reshape(_ROWS, _W)
    e2t = et.reshape(_ROWS, _W)

    nk = jnp.float32(_TOPK)
    lo = jnp.zeros((2,), jnp.int32)
    hi = jnp.full((2,), 0x7F7FFFFF, jnp.int32)
    cg_hi = jnp.zeros((2,), jnp.float32)
    sg_hi = jnp.zeros((2,), jnp.float32)
    j_idx = jnp.arange(1, _NTHR + 1, dtype=jnp.int32)

    for _ in range(_NPASS):
        step = (hi - lo) // (_NTHR + 1)
        u = lo[:, None] + step[:, None] * j_idx[None, :]
        thr = jax.lax.bitcast_convert_type(u, jnp.float32)
        cnt4, sm4 = _run_count(thr, e2p, e2t)
        cnt = jnp.sum(cnt4, axis=(2, 3))
        sm = jnp.sum(sm4, axis=(2, 3))
        ge = cnt >= nk
        new_lo = jnp.max(jnp.where(ge, u, lo[:, None]), axis=1)
        new_hi = jnp.min(jnp.where(ge, hi[:, None], u), axis=1)
        first_lt = jnp.minimum(jnp.sum(ge.astype(jnp.int32), axis=1),
                               _NTHR - 1)
        any_lt = jnp.any(~ge, axis=1)
        cnt_at = jnp.take_along_axis(cnt, first_lt[:, None], axis=1)[:, 0]
        sm_at = jnp.take_along_axis(sm, first_lt[:, None], axis=1)[:, 0]
        cg_hi = jnp.where(any_lt, cnt_at, cg_hi)
        sg_hi = jnp.where(any_lt, sm_at, sg_hi)
        lo, hi = new_lo, new_hi

    v_lo = jax.lax.bitcast_convert_type(lo, jnp.float32)
    v_hi = jax.lax.bitcast_convert_type(hi, jnp.float32)
    t_mid = 0.5 * (v_lo + v_hi)
    s_top = sg_hi + (nk - cg_hi) * t_mid
    topk_loss = jnp.abs(s_top[0] / nk - s_top[1] / nk)
    return (stats_loss + topk_loss).astype(jnp.float32)


# speedup vs baseline: 27.0051x; 1.8317x over previous
import jax
import jax.numpy as jnp
from jax.experimental import pallas as pl
from jax.experimental.pallas import tpu as pltpu

_B, _H, _W = 16, 512, 512
_N = _B * _H * _W
_TOPK = int(_N * 0.1)
_NTHR = 16
_NPASS = 6
_ROWS = _N // _W
_BLK = 512
_NBLK = _ROWS // _BLK


def _edge_stats_kernel(p_ref, t_ref, ep_ref, et_ref, acc_ref):
    i = pl.program_id(0)

    @pl.when(i == 0)
    def _():
        acc_ref[...] = jnp.zeros_like(acc_ref)

    def edges(a):
        z_row = jnp.zeros((1, _W), jnp.float32)
        up = jnp.concatenate([z_row, a[:-1, :]], axis=0)
        dn = jnp.concatenate([a[1:, :], z_row], axis=0)
        s = up + 2.0 * a + dn
        d = dn - up
        z_col = jnp.zeros((_H, 1), jnp.float32)
        ex = jnp.concatenate([s[:, 1:], z_col], axis=1) - \
            jnp.concatenate([z_col, s[:, :-1]], axis=1)
        ey = jnp.concatenate([z_col, d[:, :-1]], axis=1) + 2.0 * d + \
            jnp.concatenate([d[:, 1:], z_col], axis=1)
        return jnp.sqrt(ex * ex + ey * ey + 1e-6)

    ep = edges(p_ref[0])
    et = edges(t_ref[0])
    ep_ref[0] = ep
    et_ref[0] = et

    def lanesum(x):
        return jnp.sum(x.reshape(_H // 8, 8, _W), axis=0)

    acc_ref[0] += lanesum(ep)
    acc_ref[1] += lanesum(ep * ep)
    acc_ref[2] += lanesum(et)
    acc_ref[3] += lanesum(et * et)


def _lanesum(x):
    return jnp.sum(x.reshape(_BLK // 8, 8, _W), axis=0)


def _count_kernel(thr_ref, ep_ref, et_ref, cnt_ref):
    b = pl.program_id(0)

    @pl.when(b == 0)
    def _():
        cnt_ref[...] = jnp.zeros_like(cnt_ref)

    for i, ref in enumerate((ep_ref, et_ref)):
        x = ref[...]
        for j in range(_NTHR):
            mask = x > thr_ref[i, j]
            cnt_ref[i, j] += _lanesum(mask.astype(jnp.float32))


def _count_sum_kernel(thr_ref, ep_ref, et_ref, cnt_ref, sm_ref):
    b = pl.program_id(0)

    @pl.when(b == 0)
    def _():
        cnt_ref[...] = jnp.zeros_like(cnt_ref)
        sm_ref[...] = jnp.zeros_like(sm_ref)

    for i, ref in enumerate((ep_ref, et_ref)):
        x = ref[...]
        for j in range(_NTHR):
            mask = x > thr_ref[i, j]
            cnt_ref[i, j] += _lanesum(mask.astype(jnp.float32))
            sm_ref[i, j] += _lanesum(jnp.where(mask, x, 0.0))


def _run_edges(p, t):
    return pl.pallas_call(
        _edge_stats_kernel,
        grid=(_B,),
        in_specs=[
            pl.BlockSpec((1, _H, _W), lambda i: (i, 0, 0)),
            pl.BlockSpec((1, _H, _W), lambda i: (i, 0, 0)),
        ],
        out_specs=[
            pl.BlockSpec((1, _H, _W), lambda i: (i, 0, 0)),
            pl.BlockSpec((1, _H, _W), lambda i: (i, 0, 0)),
            pl.BlockSpec((4, 8, _W), lambda i: (0, 0, 0)),
        ],
        out_shape=[
            jax.ShapeDtypeStruct((_B, _H, _W), jnp.float32),
            jax.ShapeDtypeStruct((_B, _H, _W), jnp.float32),
            jax.ShapeDtypeStruct((4, 8, _W), jnp.float32),
        ],
    )(p, t)


def _run_count(thr, e2p, e2t, with_sums):
    body = _count_sum_kernel if with_sums else _count_kernel
    n_out = 2 if with_sums else 1
    out = pl.pallas_call(
        body,
        grid=(_NBLK,),
        in_specs=[
            pl.BlockSpec(memory_space=pltpu.SMEM),
            pl.BlockSpec((_BLK, _W), lambda b: (b, 0)),
            pl.BlockSpec((_BLK, _W), lambda b: (b, 0)),
        ],
        out_specs=[
            pl.BlockSpec((2, _NTHR, 8, _W), lambda b: (0, 0, 0, 0))
        ] * n_out,
        out_shape=[
            jax.ShapeDtypeStruct((2, _NTHR, 8, _W), jnp.float32),
        ] * n_out,
    )(thr, e2p, e2t)
    return [jnp.sum(o, axis=(2, 3)) for o in out]


def kernel(pred, target, source):
    p = pred.reshape(_B, _H, _W)
    t = target.reshape(_B, _H, _W)
    ep, et, acc = _run_edges(p, t)

    sums = jnp.sum(acc, axis=(1, 2))
    n_f = jnp.float32(_N)
    mean_p, mean_t = sums[0] / n_f, sums[2] / n_f
    var_p = (sums[1] - sums[0] * mean_p) / (n_f - 1.0)
    var_t = (sums[3] - sums[2] * mean_t) / (n_f - 1.0)
    stats_loss = jnp.abs(mean_p - mean_t) + jnp.abs(
        jnp.sqrt(var_p) - jnp.sqrt(var_t))

    e2p = ep.reshape(_ROWS, _W)
    e2t = et.reshape(_ROWS, _W)

    nk = jnp.float32(_TOPK)
    lo = jnp.zeros((2,), jnp.int32)
    hi = jnp.full((2,), 0x7F7FFFFF, jnp.int32)
    j_idx = jnp.arange(1, _NTHR + 1, dtype=jnp.int32)

    for _ in range(_NPASS - 1):
        step = (hi - lo) // (_NTHR + 1)
        u = lo[:, None] + step[:, None] * j_idx[None, :]
        thr = jax.lax.bitcast_convert_type(u, jnp.float32)
        (cnt,) = _run_count(thr, e2p, e2t, with_sums=False)
        ge = cnt >= nk
        lo = jnp.max(jnp.where(ge, u, lo[:, None]), axis=1)
        hi = jnp.min(jnp.where(ge, hi[:, None], u), axis=1)

    step = (hi - lo) // _NTHR
    u = lo[:, None] + step[:, None] * j_idx[None, :]
    u = u.at[:, _NTHR - 1].set(hi)
    thr = jax.lax.bitcast_convert_type(u, jnp.float32)
    cnt, sm = _run_count(thr, e2p, e2t, with_sums=True)
    ge = cnt >= nk
    first_lt = jnp.minimum(jnp.sum(ge.astype(jnp.int32), axis=1), _NTHR - 1)
    lo = jnp.max(jnp.where(ge, u, lo[:, None]), axis=1)
    hi = jnp.min(jnp.where(ge, hi[:, None], u), axis=1)
    cg_hi = jnp.take_along_axis(cnt, first_lt[:, None], axis=1)[:, 0]
    sg_hi = jnp.take_along_axis(sm, first_lt[:, None], axis=1)[:, 0]

    v_lo = jax.lax.bitcast_convert_type(lo, jnp.float32)
    v_hi = jax.lax.bitcast_convert_type(hi, jnp.float32)
    t_mid = 0.5 * (v_lo + v_hi)
    s_top = sg_hi + (nk - cg_hi) * t_mid
    topk_loss = jnp.abs(s_top[0] / nk - s_top[1] / nk)
    return (stats_loss + topk_loss).astype(jnp.float32)
